# Initial kernel scaffold; baseline (speedup 1.0000x reference)
import jax
import jax.numpy as jnp
from jax.experimental import pallas as pl


def kernel(x1, x2, W1, b1, g1, beta1, W2, b2, g2, beta2, W3, b3):
    raise NotImplementedError("write your pallas kernel here")


# speedup vs baseline: 6.4171x; 6.4171x over previous
import jax
import jax.numpy as jnp
from jax.experimental import pallas as pl
from jax.experimental.pallas import tpu as pltpu

_B = 8192
_D_IN = 512
_D_H = 256
_D_OUT = 128
_TEMP = 0.07
_EPS_BN = 1e-5

_BR = 1024
_NB = _B // _BR
_BRL = 2048
_NBL2 = _B // _BRL // 2


def _stats_accum(st_ref, a, j):
    st = jnp.concatenate(
        [jnp.sum(a, axis=0, keepdims=True),
         jnp.sum(a * a, axis=0, keepdims=True)], axis=0)

    @pl.when(j == 0)
    def _():
        st_ref[0] = st

    @pl.when(j != 0)
    def _():
        st_ref[0] = st_ref[0] + st


def _bn_relu(a, st, g, beta):
    mu = st[0:1] * (1.0 / _B)
    var = st[1:2] * (1.0 / _B) - mu * mu
    scale = jax.lax.rsqrt(var + _EPS_BN) * g
    return jnp.maximum((a - mu) * scale + beta, 0.0)


def _l1_body(x1_ref, x2_ref, w_ref, b_ref, a_ref, st_ref):
    h = pl.program_id(0)
    j = pl.program_id(1)

    def compute(x):
        a = jnp.dot(x, w_ref[...], preferred_element_type=jnp.float32) + b_ref[...]
        a_ref[0] = a
        _stats_accum(st_ref, a, j)

    @pl.when(h == 0)
    def _():
        compute(x1_ref[...])

    @pl.when(h == 1)
    def _():
        compute(x2_ref[...])


def _l2_body(a_ref, st_ref, g_ref, be_ref, w_ref, b_ref, o_ref, st2_ref):
    j = pl.program_id(1)
    hh = _bn_relu(a_ref[0], st_ref[0], g_ref[...], be_ref[...])
    o = jnp.dot(hh, w_ref[...], preferred_element_type=jnp.float32) + b_ref[...]
    o_ref[0] = o
    _stats_accum(st2_ref, o, j)


def _l3_body(a_ref, st_ref, g_ref, be_ref, w_ref, b_ref, z_ref, s_ref):
    j = pl.program_id(1)
    hh = _bn_relu(a_ref[0], st_ref[0], g_ref[...], be_ref[...])
    z = jnp.dot(hh, w_ref[...], preferred_element_type=jnp.float32) + b_ref[...]
    nrm = jnp.sqrt(jnp.sum(z * z, axis=1, keepdims=True))
    zn = z / jnp.maximum(nrm, 1e-12)
    z_ref[0] = zn
    ssum = jnp.sum(zn, axis=0, keepdims=True)

    @pl.when(j == 0)
    def _():
        s_ref[0] = ssum

    @pl.when(j != 0)
    def _():
        s_ref[0] = s_ref[0] + ssum


def _loss_body(z_ref, s_ref, o_ref):
    j = pl.program_id(1)
    z1 = z_ref[0]
    z2 = z_ref[1]
    s = s_ref[0] + s_ref[1]
    r1 = jnp.sum(z1 * s, axis=1, keepdims=True)
    r2 = jnp.sum(z2 * s, axis=1, keepdims=True)
    q1 = jnp.sum(z1 * z1, axis=1, keepdims=True)
    q2 = jnp.sum(z2 * z2, axis=1, keepdims=True)
    p = jnp.sum(z1 * z2, axis=1, keepdims=True)
    d1 = (r1 - q1) / _TEMP
    d2 = (r2 - q2) / _TEMP
    nom = jnp.exp(p / _TEMP)
    li = -(jnp.log(nom / d1) + jnp.log(nom / d2))
    tot = jnp.sum(li, axis=0, keepdims=True) * (0.5 / _B)
    contrib = jnp.broadcast_to(tot, (1, 1, 128))

    @pl.when(j == 0)
    def _():
        o_ref[...] = contrib

    @pl.when(j != 0)
    def _():
        o_ref[...] = o_ref[...] + contrib


def _pcall(body, grid, in_specs, out_specs, out_shape, name):
    return pl.pallas_call(
        body,
        grid=grid,
        in_specs=in_specs,
        out_specs=out_specs,
        out_shape=out_shape,
        compiler_params=pltpu.CompilerParams(
            dimension_semantics=("parallel", "arbitrary")),
        name=name,
    )


def kernel(x1, x2, W1, b1, g1, beta1, W2, b2, g2, beta2, W3, b3):
    f32 = jnp.float32
    b1r = b1.reshape(1, _D_H)
    g1r = g1.reshape(1, _D_H)
    be1r = beta1.reshape(1, _D_H)
    b2r = b2.reshape(1, _D_H)
    g2r = g2.reshape(1, _D_H)
    be2r = beta2.reshape(1, _D_H)
    b3r = b3.reshape(1, _D_OUT)

    _vec = lambda d: pl.BlockSpec((1, d), lambda h, j: (0, 0))
    _mat = lambda m, n: pl.BlockSpec((m, n), lambda h, j: (0, 0))
    _rows = lambda d: pl.BlockSpec((1, _BR, d), lambda h, j: (h, j, 0))
    _st = pl.BlockSpec((1, 2, _D_H), lambda h, j: (h, 0, 0))

    a1, st1 = _pcall(
        _l1_body, (2, _NB),
        [
            pl.BlockSpec((_BR, _D_IN), lambda h, j: ((1 - h) * j, 0)),
            pl.BlockSpec((_BR, _D_IN), lambda h, j: (h * j, 0)),
            _mat(_D_IN, _D_H),
            _vec(_D_H),
        ],
        [_rows(_D_H), _st],
        [jax.ShapeDtypeStruct((2, _B, _D_H), f32),
         jax.ShapeDtypeStruct((2, 2, _D_H), f32)],
        "simclr_l1",
    )(x1, x2, W1, b1r)

    a2, st2 = _pcall(
        _l2_body, (2, _NB),
        [_rows(_D_H), _st, _vec(_D_H), _vec(_D_H), _mat(_D_H, _D_H), _vec(_D_H)],
        [_rows(_D_H), _st],
        [jax.ShapeDtypeStruct((2, _B, _D_H), f32),
         jax.ShapeDtypeStruct((2, 2, _D_H), f32)],
        "simclr_l2",
    )(a1, st1, g1r, be1r, W2, b2r)

    z, s = _pcall(
        _l3_body, (2, _NB),
        [_rows(_D_H), _st, _vec(_D_H), _vec(_D_H), _mat(_D_H, _D_OUT), _vec(_D_OUT)],
        [_rows(_D_OUT),
         pl.BlockSpec((1, 1, _D_OUT), lambda h, j: (h, 0, 0))],
        [jax.ShapeDtypeStruct((2, _B, _D_OUT), f32),
         jax.ShapeDtypeStruct((2, 1, _D_OUT), f32)],
        "simclr_l3",
    )(a2, st2, g2r, be2r, W3, b3r)

    o = _pcall(
        _loss_body, (2, _NBL2),
        [
            pl.BlockSpec((2, _BRL, _D_OUT), lambda h, j: (0, h * _NBL2 + j, 0)),
            pl.BlockSpec((2, 1, _D_OUT), lambda h, j: (0, 0, 0)),
        ],
        pl.BlockSpec((1, 1, 128), lambda h, j: (h, 0, 0)),
        jax.ShapeDtypeStruct((2, 1, 128), f32),
        "simclr_loss",
    )(z, s)

    return o[0, 0, 0] + o[1, 0, 0]


# speedup vs baseline: 10.8090x; 1.6844x over previous
import jax
import jax.numpy as jnp
from jax.experimental import pallas as pl
from jax.experimental.pallas import tpu as pltpu

_B = 8192
_D_IN = 512
_D_H = 256
_D_OUT = 128
_TEMP = 0.07
_EPS_BN = 1e-5

_BR = 2048
_NB = _B // _BR


def _colstats(a):
    return jnp.concatenate(
        [jnp.sum(a, axis=0, keepdims=True),
         jnp.sum(a * a, axis=0, keepdims=True)], axis=0)


def _bn_relu(a, st, g, beta):
    mu = st[0:1] * (1.0 / _B)
    var = st[1:2] * (1.0 / _B) - mu * mu
    scale = jax.lax.rsqrt(var + _EPS_BN) * g
    return jnp.maximum((a - mu) * scale + beta, 0.0)


def _proj_body(x1_ref, x2_ref, w1_ref, b1_ref, g1_ref, be1_ref,
               w2_ref, b2_ref, g2_ref, be2_ref, w3_ref, b3_ref,
               o_ref, ab_s, z_s, st1_s, st2_s, s_s):
    h = pl.program_id(0)
    j = pl.program_id(1)

    @pl.when(j < _NB)
    def _():
        def compute(x):
            a = jnp.dot(x, w1_ref[...],
                        preferred_element_type=jnp.float32) + b1_ref[...]
            ab_s[j, :, 0:128] = a[:, 0:128]
            ab_s[j, :, 128:256] = a[:, 128:256]
            st = _colstats(a)

            @pl.when(j == 0)
            def _():
                st1_s[...] = st

            @pl.when(j != 0)
            def _():
                st1_s[...] = st1_s[...] + st

        @pl.when(h == 0)
        def _():
            compute(x1_ref[...])

        @pl.when(h == 1)
        def _():
            compute(x2_ref[...])

    @pl.when((j >= _NB) & (j < 2 * _NB))
    def _():
        j2 = j - _NB
        hh = _bn_relu(ab_s[j2], st1_s[...], g1_ref[...], be1_ref[...])
        o = jnp.dot(hh, w2_ref[...],
                    preferred_element_type=jnp.float32) + b2_ref[...]
        ab_s[j2, :, 0:128] = o[:, 0:128]
        ab_s[j2, :, 128:256] = o[:, 128:256]
        st = _colstats(o)

        @pl.when(j2 == 0)
        def _():
            st2_s[...] = st

        @pl.when(j2 != 0)
        def _():
            st2_s[...] = st2_s[...] + st

    @pl.when((j >= 2 * _NB) & (j < 3 * _NB))
    def _():
        j3 = j - 2 * _NB
        hh = _bn_relu(ab_s[j3], st2_s[...], g2_ref[...], be2_ref[...])
        z = jnp.dot(hh, w3_ref[...],
                    preferred_element_type=jnp.float32) + b3_ref[...]
        nrm = jnp.sqrt(jnp.sum(z * z, axis=1, keepdims=True))
        zn = z / jnp.maximum(nrm, 1e-12)
        z_s[h, j3] = zn
        ssum = jnp.sum(zn, axis=0, keepdims=True)

        @pl.when((h == 0) & (j3 == 0))
        def _():
            s_s[...] = ssum

        @pl.when((h != 0) | (j3 != 0))
        def _():
            s_s[...] = s_s[...] + ssum

    @pl.when((h == 1) & (j >= 3 * _NB))
    def _():
        j4 = j - 3 * _NB
        z1 = z_s[0, j4]
        z2 = z_s[1, j4]
        s = s_s[...]
        r1 = jnp.sum(z1 * s, axis=1, keepdims=True)
        r2 = jnp.sum(z2 * s, axis=1, keepdims=True)
        q1 = jnp.sum(z1 * z1, axis=1, keepdims=True)
        q2 = jnp.sum(z2 * z2, axis=1, keepdims=True)
        p = jnp.sum(z1 * z2, axis=1, keepdims=True)
        d1 = (r1 - q1) * (1.0 / _TEMP)
        d2 = (r2 - q2) * (1.0 / _TEMP)
        li = p * (2.0 / _TEMP) - jnp.log(d1) - jnp.log(d2)
        tot = jnp.sum(li, axis=0, keepdims=True) * (-0.5 / _B)
        contrib = jnp.broadcast_to(tot, (1, 128))

        @pl.when(j4 == 0)
        def _():
            o_ref[...] = contrib

        @pl.when(j4 != 0)
        def _():
            o_ref[...] = o_ref[...] + contrib


def kernel(x1, x2, W1, b1, g1, beta1, W2, b2, g2, beta2, W3, b3):
    f32 = jnp.float32
    _vec = lambda d: pl.BlockSpec((1, d), lambda h, j: (0, 0))
    _mat = lambda m, n: pl.BlockSpec((m, n), lambda h, j: (0, 0))

    o = pl.pallas_call(
        _proj_body,
        grid=(2, 4 * _NB),
        in_specs=[
            pl.BlockSpec((_BR, _D_IN),
                         lambda h, j: ((1 - h) * jnp.minimum(j, _NB - 1), 0)),
            pl.BlockSpec((_BR, _D_IN),
                         lambda h, j: (h * jnp.minimum(j, _NB - 1), 0)),
            _mat(_D_IN, _D_H), _vec(_D_H), _vec(_D_H), _vec(_D_H),
            _mat(_D_H, _D_H), _vec(_D_H), _vec(_D_H), _vec(_D_H),
            _mat(_D_H, _D_OUT), _vec(_D_OUT),
        ],
        out_specs=pl.BlockSpec((1, 128), lambda h, j: (0, 0)),
        out_shape=jax.ShapeDtypeStruct((1, 128), f32),
        scratch_shapes=[
            pltpu.VMEM((_NB, _BR, _D_H), f32),
            pltpu.VMEM((2, _NB, _BR, _D_OUT), f32),
            pltpu.VMEM((2, _D_H), f32),
            pltpu.VMEM((2, _D_H), f32),
            pltpu.VMEM((1, _D_OUT), f32),
        ],
        compiler_params=pltpu.CompilerParams(
            dimension_semantics=("arbitrary", "arbitrary"),
            vmem_limit_bytes=50 * 1024 * 1024,
        ),
        name="simclr_fused",
    )(x1, x2, W1, b1.reshape(1, _D_H), g1.reshape(1, _D_H),
      beta1.reshape(1, _D_H), W2, b2.reshape(1, _D_H), g2.reshape(1, _D_H),
      beta2.reshape(1, _D_H), W3, b3.reshape(1, _D_OUT))

    return o[0, 0]


# speedup vs baseline: 11.6955x; 1.0820x over previous
import jax
import jax.numpy as jnp
from jax.experimental import pallas as pl
from jax.experimental.pallas import tpu as pltpu

_B = 8192
_D_IN = 512
_D_H = 256
_D_OUT = 128
_TEMP = 0.07
_EPS_BN = 1e-5

_BR = 2048
_NB = _B // _BR


def _colstats(a):
    return jnp.concatenate(
        [jnp.sum(a, axis=0, keepdims=True),
         jnp.sum(a * a, axis=0, keepdims=True)], axis=0)


def _bn_relu(a, st, g, beta):
    mu = st[0:1] * (1.0 / _B)
    var = st[1:2] * (1.0 / _B) - mu * mu
    scale = jax.lax.rsqrt(var + _EPS_BN) * g
    shift = beta - mu * scale
    return jnp.maximum(a * scale + shift, 0.0)


def _proj_body(x1_ref, x2_ref, w1_ref, b1_ref, g1_ref, be1_ref,
               w2_ref, b2_ref, g2_ref, be2_ref, w3_ref, b3_ref,
               o_ref, ab_s, z_s, st1_s, st2_s, s_s):
    h = pl.program_id(0)
    j = pl.program_id(1)

    @pl.when(j < _NB)
    def _():
        def compute(x):
            a = jnp.dot(x, w1_ref[...],
                        preferred_element_type=jnp.float32) + b1_ref[...]
            ab_s[j, :, 0:128] = a[:, 0:128]
            ab_s[j, :, 128:256] = a[:, 128:256]
            st = _colstats(a)

            @pl.when(j == 0)
            def _():
                st1_s[...] = st

            @pl.when(j != 0)
            def _():
                st1_s[...] = st1_s[...] + st

        @pl.when(h == 0)
        def _():
            compute(x1_ref[...])

        @pl.when(h == 1)
        def _():
            compute(x2_ref[...])

    @pl.when((j >= _NB) & (j < 2 * _NB))
    def _():
        j2 = j - _NB
        hh = _bn_relu(ab_s[j2], st1_s[...], g1_ref[...], be1_ref[...])
        o = jnp.dot(hh, w2_ref[...],
                    preferred_element_type=jnp.float32) + b2_ref[...]
        ab_s[j2, :, 0:128] = o[:, 0:128]
        ab_s[j2, :, 128:256] = o[:, 128:256]
        st = _colstats(o)

        @pl.when(j2 == 0)
        def _():
            st2_s[...] = st

        @pl.when(j2 != 0)
        def _():
            st2_s[...] = st2_s[...] + st

    @pl.when((j >= 2 * _NB) & (j < 3 * _NB))
    def _():
        j3 = j - 2 * _NB
        hh = _bn_relu(ab_s[j3], st2_s[...], g2_ref[...], be2_ref[...])
        z = jnp.dot(hh, w3_ref[...],
                    preferred_element_type=jnp.float32) + b3_ref[...]
        nrm2 = jnp.sum(z * z, axis=1, keepdims=True)
        zn = z * jax.lax.rsqrt(jnp.maximum(nrm2, 1e-24))
        z_s[h, j3] = zn
        ssum = jnp.sum(zn, axis=0, keepdims=True)

        @pl.when((h == 0) & (j3 == 0))
        def _():
            s_s[...] = ssum

        @pl.when((h != 0) | (j3 != 0))
        def _():
            s_s[...] = s_s[...] + ssum

    @pl.when((h == 1) & (j >= 3 * _NB))
    def _():
        j4 = j - 3 * _NB
        z1 = z_s[0, j4]
        z2 = z_s[1, j4]
        s = s_s[...]
        m1 = jnp.sum(z1 * (s - z1), axis=1, keepdims=True)
        m2 = jnp.sum(z2 * (s - z2), axis=1, keepdims=True)
        p = jnp.sum(z1 * z2, axis=1, keepdims=True)
        d1 = m1 * (1.0 / _TEMP)
        d2 = m2 * (1.0 / _TEMP)
        li = p * (2.0 / _TEMP) - jnp.log(d1) - jnp.log(d2)
        tot = jnp.sum(li, axis=0, keepdims=True) * (-0.5 / _B)
        contrib = jnp.broadcast_to(tot, (1, 128))

        @pl.when(j4 == 0)
        def _():
            o_ref[...] = contrib

        @pl.when(j4 != 0)
        def _():
            o_ref[...] = o_ref[...] + contrib


def kernel(x1, x2, W1, b1, g1, beta1, W2, b2, g2, beta2, W3, b3):
    f32 = jnp.float32
    _vec = lambda d: pl.BlockSpec((1, d), lambda h, j: (0, 0))
    _mat = lambda m, n: pl.BlockSpec((m, n), lambda h, j: (0, 0))

    o = pl.pallas_call(
        _proj_body,
        grid=(2, 4 * _NB),
        in_specs=[
            pl.BlockSpec((_BR, _D_IN),
                         lambda h, j: ((1 - h) * jnp.minimum(j, _NB - 1), 0)),
            pl.BlockSpec((_BR, _D_IN),
                         lambda h, j: (h * jnp.minimum(j, _NB - 1), 0)),
            _mat(_D_IN, _D_H), _vec(_D_H), _vec(_D_H), _vec(_D_H),
            _mat(_D_H, _D_H), _vec(_D_H), _vec(_D_H), _vec(_D_H),
            _mat(_D_H, _D_OUT), _vec(_D_OUT),
        ],
        out_specs=pl.BlockSpec((1, 128), lambda h, j: (0, 0)),
        out_shape=jax.ShapeDtypeStruct((1, 128), f32),
        scratch_shapes=[
            pltpu.VMEM((_NB, _BR, _D_H), f32),
            pltpu.VMEM((2, _NB, _BR, _D_OUT), f32),
            pltpu.VMEM((2, _D_H), f32),
            pltpu.VMEM((2, _D_H), f32),
            pltpu.VMEM((1, _D_OUT), f32),
        ],
        compiler_params=pltpu.CompilerParams(
            dimension_semantics=("arbitrary", "arbitrary"),
            vmem_limit_bytes=50 * 1024 * 1024,
        ),
        name="simclr_fused",
    )(x1, x2, W1, b1.reshape(1, _D_H), g1.reshape(1, _D_H),
      beta1.reshape(1, _D_H), W2, b2.reshape(1, _D_H), g2.reshape(1, _D_H),
      beta2.reshape(1, _D_H), W3, b3.reshape(1, _D_OUT))

    return o[0, 0]
